# initial kernel scaffold (unmeasured)
import jax
import jax.numpy as jnp
from jax import lax
from jax.experimental import pallas as pl
from jax.experimental.pallas import tpu as pltpu


def kernel(
    u,
):
    def body(*refs):
        pass

    out_shape = jax.ShapeDtypeStruct(..., jnp.float32)
    return pl.pallas_call(body, out_shape=out_shape)(...)


# baseline (device time: 10695 ns/iter reference)
import jax
import jax.numpy as jnp
from jax import lax
from jax.experimental import pallas as pl
from jax.experimental.pallas import tpu as pltpu

NX, NY, NZ = 2, 4, 4


def kernel(u):
    SX, SY, SZ = u.shape
    GX, GY, GZ = NX * SX, NY * SY, NZ * SZ

    def body(u_ref, out_ref, hx_ref, hy_ref, hz_ref,
             ysend_ref, zsend_ref, send_sems, recv_sems):
        px = lax.axis_index("x")
        py = lax.axis_index("y")
        pz = lax.axis_index("z")

        has_xm = px > 0
        has_xp = px < NX - 1
        has_ym = py > 0
        has_yp = py < NY - 1
        has_zm = pz > 0
        has_zp = pz < NZ - 1

        hx_ref[...] = jnp.zeros_like(hx_ref)
        hy_ref[...] = jnp.zeros_like(hy_ref)
        hz_ref[...] = jnp.zeros_like(hz_ref)

        ysend_ref[0] = u_ref[:, 0, :]
        ysend_ref[1] = u_ref[:, SY - 1, :]
        zsend_ref[0] = u_ref[:, :, 0]
        zsend_ref[1] = u_ref[:, :, SZ - 1]

        barrier = pltpu.get_barrier_semaphore()
        neighbors = [
            (has_xm, (px - 1, py, pz)),
            (has_xp, (px + 1, py, pz)),
            (has_ym, (px, py - 1, pz)),
            (has_yp, (px, py + 1, pz)),
            (has_zm, (px, py, pz - 1)),
            (has_zp, (px, py, pz + 1)),
        ]
        for flag, dev in neighbors:
            @pl.when(flag)
            def _(dev=dev):
                pl.semaphore_signal(
                    barrier, inc=1,
                    device_id=dev, device_id_type=pl.DeviceIdType.MESH,
                )
        for flag, _ in neighbors:
            @pl.when(flag)
            def _():
                pl.semaphore_wait(barrier, 1)

        sends = [
            (has_xp, (px + 1, py, pz), u_ref.at[SX - 1], hx_ref.at[0], 0),
            (has_xm, (px - 1, py, pz), u_ref.at[0],      hx_ref.at[1], 1),
            (has_yp, (px, py + 1, pz), ysend_ref.at[1],  hy_ref.at[0], 2),
            (has_ym, (px, py - 1, pz), ysend_ref.at[0],  hy_ref.at[1], 3),
            (has_zp, (px, py, pz + 1), zsend_ref.at[1],  hz_ref.at[0], 4),
            (has_zm, (px, py, pz - 1), zsend_ref.at[0],  hz_ref.at[1], 5),
        ]
        rdmas = []
        for flag, dev, src, dst, idx in sends:
            rdma = pltpu.make_async_remote_copy(
                src_ref=src,
                dst_ref=dst,
                send_sem=send_sems.at[idx],
                recv_sem=recv_sems.at[idx],
                device_id=dev,
                device_id_type=pl.DeviceIdType.MESH,
            )
            rdmas.append((flag, rdma))

            @pl.when(flag)
            def _(rdma=rdma):
                rdma.start()

        recvs = [
            (has_xm, hx_ref.at[0], 0),
            (has_xp, hx_ref.at[1], 1),
            (has_ym, hy_ref.at[0], 2),
            (has_yp, hy_ref.at[1], 3),
            (has_zm, hz_ref.at[0], 4),
            (has_zp, hz_ref.at[1], 5),
        ]
        for flag, dst, idx in recvs:
            rdma = pltpu.make_async_remote_copy(
                src_ref=dst,
                dst_ref=dst,
                send_sem=send_sems.at[idx],
                recv_sem=recv_sems.at[idx],
                device_id=(px, py, pz),
                device_id_type=pl.DeviceIdType.MESH,
            )

            @pl.when(flag)
            def _(rdma=rdma):
                rdma.wait_recv()

        for flag, rdma in rdmas:
            @pl.when(flag)
            def _(rdma=rdma):
                rdma.wait_send()

        uv = u_ref[...]
        xm = jnp.concatenate([hx_ref[0][None], uv[:-1]], axis=0)
        xp = jnp.concatenate([uv[1:], hx_ref[1][None]], axis=0)
        ym = jnp.concatenate([hy_ref[0][:, None, :], uv[:, :-1, :]], axis=1)
        yp = jnp.concatenate([uv[:, 1:, :], hy_ref[1][:, None, :]], axis=1)
        zm = jnp.concatenate([hz_ref[0][:, :, None], uv[:, :, :-1]], axis=2)
        zp = jnp.concatenate([uv[:, :, 1:], hz_ref[1][:, :, None]], axis=2)
        v = xm + xp + ym + yp + zm + zp - 6.0 * uv

        gx = lax.broadcasted_iota(jnp.int32, (SX, SY, SZ), 0) + px * SX
        gy = lax.broadcasted_iota(jnp.int32, (SX, SY, SZ), 1) + py * SY
        gz = lax.broadcasted_iota(jnp.int32, (SX, SY, SZ), 2) + pz * SZ
        bnd = (
            (gx == 0) | (gx == GX - 1)
            | (gy == 0) | (gy == GY - 1)
            | (gz == 0) | (gz == GZ - 1)
        )
        out_ref[...] = jnp.where(bnd, 0.0, v)

    return pl.pallas_call(
        body,
        out_shape=jax.ShapeDtypeStruct((SX, SY, SZ), u.dtype),
        in_specs=[pl.BlockSpec(memory_space=pltpu.VMEM)],
        out_specs=pl.BlockSpec(memory_space=pltpu.VMEM),
        scratch_shapes=[
            pltpu.VMEM((2, SY, SZ), u.dtype),
            pltpu.VMEM((2, SX, SZ), u.dtype),
            pltpu.VMEM((2, SX, SY), u.dtype),
            pltpu.VMEM((2, SX, SZ), u.dtype),
            pltpu.VMEM((2, SX, SY), u.dtype),
            pltpu.SemaphoreType.DMA((6,)),
            pltpu.SemaphoreType.DMA((6,)),
        ],
        compiler_params=pltpu.CompilerParams(collective_id=0),
    )(u)


# device time: 9431 ns/iter; 1.1340x vs baseline; 1.1340x over previous
import jax
import jax.numpy as jnp
from jax import lax
from jax.experimental import pallas as pl
from jax.experimental.pallas import tpu as pltpu

NX, NY, NZ = 2, 4, 4


def kernel(u):
    SX, SY, SZ = u.shape
    GX, GY, GZ = NX * SX, NY * SY, NZ * SZ

    def body(u_ref, out_ref, hx_ref, hy_ref, hz_ref,
             ysend_ref, zsend_ref, send_sems, recv_sems):
        px = lax.axis_index("x")
        py = lax.axis_index("y")
        pz = lax.axis_index("z")

        has_xm = px > 0
        has_xp = px < NX - 1
        has_ym = py > 0
        has_yp = py < NY - 1
        has_zm = pz > 0
        has_zp = pz < NZ - 1

        ysend_ref[0] = u_ref[:, 0, :]
        ysend_ref[1] = u_ref[:, SY - 1, :]
        zsend_ref[0] = u_ref[:, :, 0]
        zsend_ref[1] = u_ref[:, :, SZ - 1]

        barrier = pltpu.get_barrier_semaphore()
        neighbors = [
            (has_xm, (px - 1, py, pz)),
            (has_xp, (px + 1, py, pz)),
            (has_ym, (px, py - 1, pz)),
            (has_yp, (px, py + 1, pz)),
            (has_zm, (px, py, pz - 1)),
            (has_zp, (px, py, pz + 1)),
        ]
        for flag, dev in neighbors:
            @pl.when(flag)
            def _(dev=dev):
                pl.semaphore_signal(
                    barrier, inc=1,
                    device_id=dev, device_id_type=pl.DeviceIdType.MESH,
                )
        for flag, _ in neighbors:
            @pl.when(flag)
            def _():
                pl.semaphore_wait(barrier, 1)

        sends = [
            (has_xp, (px + 1, py, pz), u_ref.at[SX - 1], hx_ref.at[0], 0),
            (has_xm, (px - 1, py, pz), u_ref.at[0],      hx_ref.at[1], 1),
            (has_yp, (px, py + 1, pz), ysend_ref.at[1],  hy_ref.at[0], 2),
            (has_ym, (px, py - 1, pz), ysend_ref.at[0],  hy_ref.at[1], 3),
            (has_zp, (px, py, pz + 1), zsend_ref.at[1],  hz_ref.at[0], 4),
            (has_zm, (px, py, pz - 1), zsend_ref.at[0],  hz_ref.at[1], 5),
        ]
        rdmas = []
        for flag, dev, src, dst, idx in sends:
            rdma = pltpu.make_async_remote_copy(
                src_ref=src,
                dst_ref=dst,
                send_sem=send_sems.at[idx],
                recv_sem=recv_sems.at[idx],
                device_id=dev,
                device_id_type=pl.DeviceIdType.MESH,
            )
            rdmas.append((flag, rdma))

            @pl.when(flag)
            def _(rdma=rdma):
                rdma.start()

        uv = u_ref[...]
        zyz = jnp.zeros((1, SY, SZ), uv.dtype)
        zxz = jnp.zeros((SX, 1, SZ), uv.dtype)
        zxy = jnp.zeros((SX, SY, 1), uv.dtype)
        xm = jnp.concatenate([zyz, uv[:-1]], axis=0)
        xp = jnp.concatenate([uv[1:], zyz], axis=0)
        ym = jnp.concatenate([zxz, uv[:, :-1, :]], axis=1)
        yp = jnp.concatenate([uv[:, 1:, :], zxz], axis=1)
        zm = jnp.concatenate([zxy, uv[:, :, :-1]], axis=2)
        zp = jnp.concatenate([uv[:, :, 1:], zxy], axis=2)
        v = xm + xp + ym + yp + zm + zp - 6.0 * uv

        gx = lax.broadcasted_iota(jnp.int32, (SX, SY, SZ), 0) + px * SX
        gy = lax.broadcasted_iota(jnp.int32, (SX, SY, SZ), 1) + py * SY
        gz = lax.broadcasted_iota(jnp.int32, (SX, SY, SZ), 2) + pz * SZ
        bnd = (
            (gx == 0) | (gx == GX - 1)
            | (gy == 0) | (gy == GY - 1)
            | (gz == 0) | (gz == GZ - 1)
        )
        out_ref[...] = jnp.where(bnd, 0.0, v)

        recvs = [
            (has_xm, hx_ref.at[0], 0),
            (has_xp, hx_ref.at[1], 1),
            (has_ym, hy_ref.at[0], 2),
            (has_yp, hy_ref.at[1], 3),
            (has_zm, hz_ref.at[0], 4),
            (has_zp, hz_ref.at[1], 5),
        ]
        for flag, dst, idx in recvs:
            rdma = pltpu.make_async_remote_copy(
                src_ref=dst,
                dst_ref=dst,
                send_sem=send_sems.at[idx],
                recv_sem=recv_sems.at[idx],
                device_id=(px, py, pz),
                device_id_type=pl.DeviceIdType.MESH,
            )

            @pl.when(flag)
            def _(rdma=rdma):
                rdma.wait_recv()

        gy2 = lax.broadcasted_iota(jnp.int32, (SY, SZ), 0) + py * SY
        gz2 = lax.broadcasted_iota(jnp.int32, (SY, SZ), 1) + pz * SZ
        mask_x = (gy2 == 0) | (gy2 == GY - 1) | (gz2 == 0) | (gz2 == GZ - 1)
        gx2 = lax.broadcasted_iota(jnp.int32, (SX, SZ), 0) + px * SX
        gz3 = lax.broadcasted_iota(jnp.int32, (SX, SZ), 1) + pz * SZ
        mask_y = (gx2 == 0) | (gx2 == GX - 1) | (gz3 == 0) | (gz3 == GZ - 1)
        gx3 = lax.broadcasted_iota(jnp.int32, (SX, SY), 0) + px * SX
        gy3 = lax.broadcasted_iota(jnp.int32, (SX, SY), 1) + py * SY
        mask_z = (gx3 == 0) | (gx3 == GX - 1) | (gy3 == 0) | (gy3 == GY - 1)

        @pl.when(has_xm)
        def _():
            out_ref[0, :, :] = jnp.where(
                mask_x, 0.0, out_ref[0, :, :] + hx_ref[0])

        @pl.when(has_xp)
        def _():
            out_ref[SX - 1, :, :] = jnp.where(
                mask_x, 0.0, out_ref[SX - 1, :, :] + hx_ref[1])

        @pl.when(has_ym)
        def _():
            out_ref[:, 0, :] = jnp.where(
                mask_y, 0.0, out_ref[:, 0, :] + hy_ref[0])

        @pl.when(has_yp)
        def _():
            out_ref[:, SY - 1, :] = jnp.where(
                mask_y, 0.0, out_ref[:, SY - 1, :] + hy_ref[1])

        @pl.when(has_zm)
        def _():
            out_ref[:, :, 0] = jnp.where(
                mask_z, 0.0, out_ref[:, :, 0] + hz_ref[0])

        @pl.when(has_zp)
        def _():
            out_ref[:, :, SZ - 1] = jnp.where(
                mask_z, 0.0, out_ref[:, :, SZ - 1] + hz_ref[1])

        for flag, rdma in rdmas:
            @pl.when(flag)
            def _(rdma=rdma):
                rdma.wait_send()

    return pl.pallas_call(
        body,
        out_shape=jax.ShapeDtypeStruct((SX, SY, SZ), u.dtype),
        in_specs=[pl.BlockSpec(memory_space=pltpu.VMEM)],
        out_specs=pl.BlockSpec(memory_space=pltpu.VMEM),
        scratch_shapes=[
            pltpu.VMEM((2, SY, SZ), u.dtype),
            pltpu.VMEM((2, SX, SZ), u.dtype),
            pltpu.VMEM((2, SX, SY), u.dtype),
            pltpu.VMEM((2, SX, SZ), u.dtype),
            pltpu.VMEM((2, SX, SY), u.dtype),
            pltpu.SemaphoreType.DMA((6,)),
            pltpu.SemaphoreType.DMA((6,)),
        ],
        compiler_params=pltpu.CompilerParams(collective_id=0),
    )(u)


# device time: 8898 ns/iter; 1.2020x vs baseline; 1.0599x over previous
import jax
import jax.numpy as jnp
from jax import lax
from jax.experimental import pallas as pl
from jax.experimental.pallas import tpu as pltpu

NX, NY, NZ = 2, 4, 4


def kernel(u):
    SX, SY, SZ = u.shape
    GX, GY, GZ = NX * SX, NY * SY, NZ * SZ

    def body(u_ref, out_ref, hx_ref, hy_ref, hz_ref,
             ysend_ref, zsend_ref, send_sems, recv_sems):
        px = lax.axis_index("x")
        py = lax.axis_index("y")
        pz = lax.axis_index("z")

        has_xm = px > 0
        has_xp = px < NX - 1
        has_ym = py > 0
        has_yp = py < NY - 1
        has_zm = pz > 0
        has_zp = pz < NZ - 1

        ysend_ref[0] = u_ref[:, 0, :]
        ysend_ref[1] = u_ref[:, SY - 1, :]
        zsend_ref[0] = u_ref[:, :, 0]
        zsend_ref[1] = u_ref[:, :, SZ - 1]

        barrier = pltpu.get_barrier_semaphore()
        neighbors = [
            (has_xm, (px - 1, py, pz)),
            (has_xp, (px + 1, py, pz)),
            (has_ym, (px, py - 1, pz)),
            (has_yp, (px, py + 1, pz)),
            (has_zm, (px, py, pz - 1)),
            (has_zp, (px, py, pz + 1)),
        ]
        for flag, dev in neighbors:
            @pl.when(flag)
            def _(dev=dev):
                pl.semaphore_signal(
                    barrier, inc=1,
                    device_id=dev, device_id_type=pl.DeviceIdType.MESH,
                )
        for flag, _ in neighbors:
            @pl.when(flag)
            def _():
                pl.semaphore_wait(barrier, 1)

        sends = [
            (has_xp, (px + 1, py, pz), u_ref.at[SX - 1], hx_ref.at[0], 0),
            (has_xm, (px - 1, py, pz), u_ref.at[0],      hx_ref.at[1], 1),
            (has_yp, (px, py + 1, pz), ysend_ref.at[1],  hy_ref.at[0], 2),
            (has_ym, (px, py - 1, pz), ysend_ref.at[0],  hy_ref.at[1], 3),
            (has_zp, (px, py, pz + 1), zsend_ref.at[1],  hz_ref.at[0], 4),
            (has_zm, (px, py, pz - 1), zsend_ref.at[0],  hz_ref.at[1], 5),
        ]
        rdmas = []
        for flag, dev, src, dst, idx in sends:
            rdma = pltpu.make_async_remote_copy(
                src_ref=src,
                dst_ref=dst,
                send_sem=send_sems.at[idx],
                recv_sem=recv_sems.at[idx],
                device_id=dev,
                device_id_type=pl.DeviceIdType.MESH,
            )
            rdmas.append((flag, rdma))

            @pl.when(flag)
            def _(rdma=rdma):
                rdma.start()

        uv = u_ref[...]
        zyz = jnp.zeros((1, SY, SZ), uv.dtype)
        zxz = jnp.zeros((SX, 1, SZ), uv.dtype)
        zxy = jnp.zeros((SX, SY, 1), uv.dtype)
        xm = jnp.concatenate([zyz, uv[:-1]], axis=0)
        xp = jnp.concatenate([uv[1:], zyz], axis=0)
        ym = jnp.concatenate([zxz, uv[:, :-1, :]], axis=1)
        yp = jnp.concatenate([uv[:, 1:, :], zxz], axis=1)
        zm = jnp.concatenate([zxy, uv[:, :, :-1]], axis=2)
        zp = jnp.concatenate([uv[:, :, 1:], zxy], axis=2)
        v = xm + xp + ym + yp + zm + zp - 6.0 * uv
        out_ref[...] = v

        recvs = [
            (has_xm, hx_ref.at[0], 0),
            (has_xp, hx_ref.at[1], 1),
            (has_ym, hy_ref.at[0], 2),
            (has_yp, hy_ref.at[1], 3),
            (has_zm, hz_ref.at[0], 4),
            (has_zp, hz_ref.at[1], 5),
        ]
        for flag, dst, idx in recvs:
            rdma = pltpu.make_async_remote_copy(
                src_ref=dst,
                dst_ref=dst,
                send_sem=send_sems.at[idx],
                recv_sem=recv_sems.at[idx],
                device_id=(px, py, pz),
                device_id_type=pl.DeviceIdType.MESH,
            )

            @pl.when(flag)
            def _(rdma=rdma):
                rdma.wait_recv()

        @pl.when(has_xm)
        def _():
            out_ref[0, :, :] = out_ref[0, :, :] + hx_ref[0]

        @pl.when(has_xp)
        def _():
            out_ref[SX - 1, :, :] = out_ref[SX - 1, :, :] + hx_ref[1]

        @pl.when(has_ym)
        def _():
            out_ref[:, 0, :] = out_ref[:, 0, :] + hy_ref[0]

        @pl.when(has_yp)
        def _():
            out_ref[:, SY - 1, :] = out_ref[:, SY - 1, :] + hy_ref[1]

        @pl.when(has_zm)
        def _():
            out_ref[:, :, 0] = out_ref[:, :, 0] + hz_ref[0]

        @pl.when(has_zp)
        def _():
            out_ref[:, :, SZ - 1] = out_ref[:, :, SZ - 1] + hz_ref[1]

        @pl.when(jnp.logical_not(has_xm))
        def _():
            out_ref[0, :, :] = jnp.zeros((SY, SZ), uv.dtype)

        @pl.when(jnp.logical_not(has_xp))
        def _():
            out_ref[SX - 1, :, :] = jnp.zeros((SY, SZ), uv.dtype)

        @pl.when(jnp.logical_not(has_ym))
        def _():
            out_ref[:, 0, :] = jnp.zeros((SX, SZ), uv.dtype)

        @pl.when(jnp.logical_not(has_yp))
        def _():
            out_ref[:, SY - 1, :] = jnp.zeros((SX, SZ), uv.dtype)

        @pl.when(jnp.logical_not(has_zm))
        def _():
            out_ref[:, :, 0] = jnp.zeros((SX, SY), uv.dtype)

        @pl.when(jnp.logical_not(has_zp))
        def _():
            out_ref[:, :, SZ - 1] = jnp.zeros((SX, SY), uv.dtype)

        for flag, rdma in rdmas:
            @pl.when(flag)
            def _(rdma=rdma):
                rdma.wait_send()

    return pl.pallas_call(
        body,
        out_shape=jax.ShapeDtypeStruct((SX, SY, SZ), u.dtype),
        in_specs=[pl.BlockSpec(memory_space=pltpu.VMEM)],
        out_specs=pl.BlockSpec(memory_space=pltpu.VMEM),
        scratch_shapes=[
            pltpu.VMEM((2, SY, SZ), u.dtype),
            pltpu.VMEM((2, SX, SZ), u.dtype),
            pltpu.VMEM((2, SX, SY), u.dtype),
            pltpu.VMEM((2, SX, SZ), u.dtype),
            pltpu.VMEM((2, SX, SY), u.dtype),
            pltpu.SemaphoreType.DMA((6,)),
            pltpu.SemaphoreType.DMA((6,)),
        ],
        compiler_params=pltpu.CompilerParams(collective_id=0),
    )(u)
